# baseline (speedup 1.0000x reference)
import functools

import jax
import jax.numpy as jnp
import numpy as np
from jax import lax
from jax.experimental import pallas as pl
from jax.experimental.pallas import tpu as pltpu
from jax.experimental.pallas import tpu_sc as plsc

N = 10000
NPAD = 10240
NC, NS = 2, 16
EPW = 10240
EPAD = NC * NS * EPW
EROWS = EPAD // 128
RW = EPW // 128
GC = 8
NG = RW // GC
ZW = NPAD // NS
ZROWS = 16

_NPAD_EDGES = EPAD - 320000
_PAD_EDGES = np.stack([
    (np.arange(_NPAD_EDGES, dtype=np.int32) % N).reshape(-1, 128),
    (np.arange(_NPAD_EDGES, dtype=np.int32) % (NPAD - N) + N).reshape(-1, 128),
])

_MESH = plsc.VectorSubcoreMesh(
    core_axis_name="c", subcore_axis_name="s", num_cores=NC, num_subcores=NS)


def _fill_1d(ref, n, value):
  def zf(k, _):
    ref[pl.ds(k * 16, 16)] = jnp.full((16,), value, jnp.float32)
    return 0
  lax.fori_loop(0, n // 16, zf, 0)


def _fill_rows(ref, rows, width, value):
  def zf(k, _):
    r = k // (width // 16)
    col = (k % (width // 16)) * 16
    ref[r, pl.ds(col, 16)] = jnp.full((16,), value, jnp.float32)
    return 0
  lax.fori_loop(0, rows * width // 16, zf, 0)


def _deg_body(edges_hbm, out_hbm, slab_v, ones_v, zbuf_v, deg_sh):
  c = lax.axis_index("c")
  s = lax.axis_index("s")
  w = c * NS + s
  _fill_1d(ones_v, 128, 1.0)
  _fill_1d(zbuf_v, ZW, 0.0)
  pltpu.sync_copy(edges_hbm.at[1, pl.ds(w * RW, RW)], slab_v)
  pltpu.sync_copy(zbuf_v, deg_sh.at[pl.ds(s * ZW, ZW)])
  plsc.subcore_barrier()
  def step(j, _):
    pltpu.sync_copy(ones_v, deg_sh.at[slab_v.at[j]], add=True)
    return 0
  lax.fori_loop(0, RW, step, 0)
  plsc.subcore_barrier()
  pltpu.sync_copy(deg_sh.at[pl.ds(s * ZW, ZW)],
                  out_hbm.at[c, pl.ds(s * ZW, ZW)])


_deg_kernel = functools.partial(
    pl.kernel,
    out_type=jax.ShapeDtypeStruct((NC, NPAD), jnp.float32),
    mesh=_MESH,
    scratch_types=[
        pltpu.VMEM((RW, 128), jnp.int32),
        pltpu.VMEM((128,), jnp.float32),
        pltpu.VMEM((ZW,), jnp.float32),
        pltpu.VMEM_SHARED((NPAD,), jnp.float32),
    ],
)(_deg_body)


def _make_scatter_kernel(D, B, K, GCB):
  CH = EPW // B
  NGB = CH // GCB
  assert GCB % (2 * K) == 0

  def body(hs_hbm, idx_hbm, out_hbm, *scratch):
    (si0, di0, si1, di1), scratch = scratch[:4], scratch[4:]
    abufs, scratch = scratch[:K], scratch[K:]
    bbufs, scratch = scratch[:K], scratch[K:]
    (zbuf, isem0, isem1), scratch = scratch[:3], scratch[3:]
    asems, scratch = scratch[:K], scratch[K:]
    bsems, scratch = scratch[:K], scratch[K:]
    sasems, scratch = scratch[:K], scratch[K:]
    sbsems, scratch = scratch[:K], scratch[K:]
    (agg_sh,) = scratch
    c = lax.axis_index("c")
    s = lax.axis_index("s")
    cbase = (c * NS + s) * CH
    zb = s * ZW

    def idx_start(gidx, sbuf, dbuf, sem):
      rows = pl.ds(cbase + gidx * GCB, GCB)
      pltpu.async_copy(idx_hbm.at[0, rows], sbuf, sem)
      pltpu.async_copy(idx_hbm.at[1, rows], dbuf, sem)

    idx_start(0, si0, di0, isem0)
    _fill_rows(zbuf, ZROWS, D, 0.0)
    def zc(t, _):
      pltpu.sync_copy(zbuf, agg_sh.at[pl.ds(zb + t * ZROWS, ZROWS)])
      return 0
    lax.fori_loop(0, ZW // ZROWS, zc, 0)
    plsc.subcore_barrier()

    def idx_wait(sbuf, dbuf, sem):
      pltpu.make_async_copy(idx_hbm.at[0, pl.ds(0, GCB)], sbuf, sem).wait()
      pltpu.make_async_copy(idx_hbm.at[1, pl.ds(0, GCB)], dbuf, sem).wait()

    def gather(row_ref, buf, sem):
      return pltpu.async_copy(hs_hbm.at[row_ref], buf, sem)

    def scat(buf, row_ref, sem):
      return pltpu.async_copy(buf, agg_sh.at[row_ref], sem, add=True)

    def gwait(buf, sem):
      pltpu.make_async_copy(hs_hbm.at[si0.at[0]], buf, sem).wait()

    def group(sbuf, dbuf, sem):
      idx_wait(sbuf, dbuf, sem)
      for k in range(K):
        gather(sbuf.at[k], abufs[k], asems[k])
      def it(t, _):
        j = 2 * K * t
        for k in range(K):
          gather(sbuf.at[j + K + k], bbufs[k], bsems[k])
        cas = []
        for k in range(K):
          gwait(abufs[k], asems[k])
          cas.append(scat(abufs[k], dbuf.at[j + k], sasems[k]))
        cbs = []
        for k in range(K):
          gwait(bbufs[k], bsems[k])
          cbs.append(scat(bbufs[k], dbuf.at[j + K + k], sbsems[k]))
        for ca in cas:
          ca.wait()
        jn = jnp.minimum(j + 2 * K, GCB - K)
        for k in range(K):
          gather(sbuf.at[jn + k], abufs[k], asems[k])
        for cb in cbs:
          cb.wait()
        return 0
      lax.fori_loop(0, GCB // (2 * K), it, 0)
      for k in range(K):
        gwait(abufs[k], asems[k])

    def pair(p, _):
      g0 = 2 * p
      idx_start(g0 + 1, si1, di1, isem1)
      group(si0, di0, isem0)
      idx_start(jnp.minimum(g0 + 2, NGB - 1), si0, di0, isem0)
      group(si1, di1, isem1)
      return 0
    lax.fori_loop(0, NGB // 2, pair, 0)
    if NGB % 2:
      group(si0, di0, isem0)
    else:
      idx_wait(si0, di0, isem0)
    plsc.subcore_barrier()
    pltpu.sync_copy(agg_sh.at[pl.ds(zb, ZW)],
                    out_hbm.at[c, pl.ds(zb, ZW)])

  return functools.partial(
      pl.kernel,
      out_type=jax.ShapeDtypeStruct((NC, NPAD, D), jnp.float32),
      mesh=_MESH,
      compiler_params=pltpu.CompilerParams(use_tc_tiling_on_sc=(D == 128)),
      scratch_types=(
          [pltpu.VMEM((GCB, B), jnp.int32)] * 4
          + [pltpu.VMEM((B, D), jnp.float32)] * (2 * K)
          + [pltpu.VMEM((ZROWS, D), jnp.float32)]
          + [pltpu.SemaphoreType.DMA] * (2 + 4 * K)
          + [pltpu.VMEM_SHARED((NPAD, D), jnp.float32)]
      ),
  )(body)


_scatter128 = _make_scatter_kernel(128, 64, 2, 16)
_scatter64 = _make_scatter_kernel(64, 128, 2, 20)


def _tc1a_body(feat_ref, w1_ref, h_ref):
  h_ref[...] = jnp.dot(feat_ref[...], w1_ref[...],
                       preferred_element_type=jnp.float32)


def _tc1b_body(h_ref, degp_ref, hs_ref, dinv_ref):
  degt = jnp.transpose(degp_ref[...])
  deg = degt[:, 0:1] + degt[:, 1:2] + 1.0
  dinv = lax.rsqrt(deg)
  hs_ref[:N, :] = dinv[:N] * h_ref[...]
  hs_ref[N:, :] = jnp.zeros((NPAD - N, 128), jnp.float32)
  dinv_ref[...] = dinv


def _tc2_body(aggp_ref, hs1_ref, dinv_ref, b1_ref, w2_ref, hs2_ref):
  agg = aggp_ref[0, :, :] + aggp_ref[1, :, :] + hs1_ref[...]
  x1 = jnp.maximum(dinv_ref[...] * agg + b1_ref[...], 0.0)
  h2 = jnp.dot(x1, w2_ref[...], preferred_element_type=jnp.float32)
  hs2_ref[...] = dinv_ref[...] * h2


def _tc3_body(aggp_ref, hs2_ref, dinv_ref, b2_ref, out_ref):
  agg = aggp_ref[0, :, :] + aggp_ref[1, :, :] + hs2_ref[...]
  z = (dinv_ref[...] * agg + b2_ref[...])[:N, :]
  m = jnp.max(z, axis=1, keepdims=True)
  e = jnp.exp(z - m)
  lse = m + jnp.log(jnp.sum(e, axis=1, keepdims=True))
  out_ref[...] = z - lse


def kernel(feature, adj, W1, b1, W2, b2):
  edges = jnp.concatenate(
      [adj.reshape(2, -1, 128), jnp.asarray(_PAD_EDGES)], axis=1)

  h1 = pl.pallas_call(
      _tc1a_body,
      out_shape=jax.ShapeDtypeStruct((N, 128), jnp.float32),
  )(feature, W1)

  deg_parts = _deg_kernel(edges)

  hs1, dinv = pl.pallas_call(
      _tc1b_body,
      out_shape=[
          jax.ShapeDtypeStruct((NPAD, 128), jnp.float32),
          jax.ShapeDtypeStruct((NPAD, 1), jnp.float32),
      ],
  )(h1, deg_parts)

  agg1 = _scatter128(hs1, edges.reshape(2, -1, 64))

  hs2 = pl.pallas_call(
      _tc2_body,
      out_shape=jax.ShapeDtypeStruct((NPAD, 64), jnp.float32),
  )(agg1, hs1, dinv, b1.reshape(1, -1), W2)

  agg2 = _scatter64(hs2, edges)

  out = pl.pallas_call(
      _tc3_body,
      out_shape=jax.ShapeDtypeStruct((N, 64), jnp.float32),
  )(agg2, hs2, dinv, b2.reshape(1, -1))
  return out


# speedup vs baseline: 1.0687x; 1.0687x over previous
import functools

import jax
import jax.numpy as jnp
import numpy as np
from jax import lax
from jax.experimental import pallas as pl
from jax.experimental.pallas import tpu as pltpu
from jax.experimental.pallas import tpu_sc as plsc

N = 10000
NPAD = 10240
NC, NS = 2, 16
EPW = 10240
EPAD = NC * NS * EPW
EROWS = EPAD // 128
RW = EPW // 128
GC = 8
NG = RW // GC
ZW = NPAD // NS
ZROWS = 16

_NPAD_EDGES = EPAD - 320000
_PAD_EDGES = np.stack([
    (np.arange(_NPAD_EDGES, dtype=np.int32) % N).reshape(-1, 128),
    (np.arange(_NPAD_EDGES, dtype=np.int32) % (NPAD - N) + N).reshape(-1, 128),
])

_MESH = plsc.VectorSubcoreMesh(
    core_axis_name="c", subcore_axis_name="s", num_cores=NC, num_subcores=NS)


def _fill_1d(ref, n, value):
  def zf(k, _):
    ref[pl.ds(k * 16, 16)] = jnp.full((16,), value, jnp.float32)
    return 0
  lax.fori_loop(0, n // 16, zf, 0)


def _fill_rows(ref, rows, width, value):
  def zf(k, _):
    r = k // (width // 16)
    col = (k % (width // 16)) * 16
    ref[r, pl.ds(col, 16)] = jnp.full((16,), value, jnp.float32)
    return 0
  lax.fori_loop(0, rows * width // 16, zf, 0)


def _deg_body(edges_hbm, out_hbm, slab_v, ones_v, zbuf_v, deg_sh):
  c = lax.axis_index("c")
  s = lax.axis_index("s")
  w = c * NS + s
  _fill_1d(ones_v, 128, 1.0)
  _fill_1d(zbuf_v, ZW, 0.0)
  pltpu.sync_copy(edges_hbm.at[1, pl.ds(w * RW, RW)], slab_v)
  pltpu.sync_copy(zbuf_v, deg_sh.at[pl.ds(s * ZW, ZW)])
  plsc.subcore_barrier()
  def step(j, _):
    pltpu.sync_copy(ones_v, deg_sh.at[slab_v.at[j]], add=True)
    return 0
  lax.fori_loop(0, RW, step, 0)
  plsc.subcore_barrier()
  pltpu.sync_copy(deg_sh.at[pl.ds(s * ZW, ZW)],
                  out_hbm.at[c, pl.ds(s * ZW, ZW)])


_deg_kernel = functools.partial(
    pl.kernel,
    out_type=jax.ShapeDtypeStruct((NC, NPAD), jnp.float32),
    mesh=_MESH,
    scratch_types=[
        pltpu.VMEM((RW, 128), jnp.int32),
        pltpu.VMEM((128,), jnp.float32),
        pltpu.VMEM((ZW,), jnp.float32),
        pltpu.VMEM_SHARED((NPAD,), jnp.float32),
    ],
)(_deg_body)


def _make_scatter_kernel(D, B, K, GCB):
  CH = EPW // B
  NGB = CH // GCB
  assert GCB % (2 * K) == 0

  def body(hs_hbm, idx_hbm, out_hbm, *scratch):
    (si0, di0, si1, di1), scratch = scratch[:4], scratch[4:]
    abufs, scratch = scratch[:K], scratch[K:]
    bbufs, scratch = scratch[:K], scratch[K:]
    (zbuf, isem0, isem1), scratch = scratch[:3], scratch[3:]
    asems, scratch = scratch[:K], scratch[K:]
    bsems, scratch = scratch[:K], scratch[K:]
    sasems, scratch = scratch[:K], scratch[K:]
    sbsems, scratch = scratch[:K], scratch[K:]
    (agg_sh,) = scratch
    c = lax.axis_index("c")
    s = lax.axis_index("s")
    cbase = (c * NS + s) * CH
    zb = s * ZW

    def idx_start(gidx, sbuf, dbuf, sem):
      rows = pl.ds(cbase + gidx * GCB, GCB)
      pltpu.async_copy(idx_hbm.at[0, rows], sbuf, sem)
      pltpu.async_copy(idx_hbm.at[1, rows], dbuf, sem)

    idx_start(0, si0, di0, isem0)
    _fill_rows(zbuf, ZROWS, D, 0.0)
    def zc(t, _):
      pltpu.sync_copy(zbuf, agg_sh.at[pl.ds(zb + t * ZROWS, ZROWS)])
      return 0
    lax.fori_loop(0, ZW // ZROWS, zc, 0)
    plsc.subcore_barrier()

    def idx_wait(sbuf, dbuf, sem):
      pltpu.make_async_copy(idx_hbm.at[0, pl.ds(0, GCB)], sbuf, sem).wait()
      pltpu.make_async_copy(idx_hbm.at[1, pl.ds(0, GCB)], dbuf, sem).wait()

    def gather(row_ref, buf, sem):
      return pltpu.async_copy(hs_hbm.at[row_ref], buf, sem)

    def scat(buf, row_ref, sem):
      return pltpu.async_copy(buf, agg_sh.at[row_ref], sem, add=True)

    def gwait(buf, sem):
      pltpu.make_async_copy(hs_hbm.at[si0.at[0]], buf, sem).wait()

    def prime(sbuf):
      for k in range(K):
        gather(sbuf.at[k], abufs[k], asems[k])

    def block(sbuf, dbuf, j, prefetch):
      for k in range(K):
        gather(sbuf.at[j + K + k], bbufs[k], bsems[k])
      cas = []
      for k in range(K):
        gwait(abufs[k], asems[k])
        cas.append(scat(abufs[k], dbuf.at[j + k], sasems[k]))
      cbs = []
      for k in range(K):
        gwait(bbufs[k], bsems[k])
        cbs.append(scat(bbufs[k], dbuf.at[j + K + k], sbsems[k]))
      for ca in cas:
        ca.wait()
      prefetch(j)
      for cb in cbs:
        cb.wait()

    def group(sbuf, dbuf, nsbuf, ndbuf, nsem):
      def it(t, _):
        block(sbuf, dbuf, 2 * K * t,
              lambda j: prime_rows(sbuf, j + 2 * K))
        return 0
      lax.fori_loop(0, GCB // (2 * K) - 1, it, 0)
      def cross(_j):
        idx_wait(nsbuf, ndbuf, nsem)
        prime(nsbuf)
      block(sbuf, dbuf, GCB - 2 * K, cross)

    def prime_rows(sbuf, j):
      for k in range(K):
        gather(sbuf.at[j + k], abufs[k], asems[k])

    idx_wait(si0, di0, isem0)
    prime(si0)
    def pair(p, _):
      g0 = 2 * p
      idx_start(g0 + 1, si1, di1, isem1)
      group(si0, di0, si1, di1, isem1)
      idx_start(jnp.minimum(g0 + 2, NGB - 1), si0, di0, isem0)
      group(si1, di1, si0, di0, isem0)
      return 0
    lax.fori_loop(0, NGB // 2, pair, 0)
    for k in range(K):
      gwait(abufs[k], asems[k])
    plsc.subcore_barrier()
    pltpu.sync_copy(agg_sh.at[pl.ds(zb, ZW)],
                    out_hbm.at[c, pl.ds(zb, ZW)])

  return functools.partial(
      pl.kernel,
      out_type=jax.ShapeDtypeStruct((NC, NPAD, D), jnp.float32),
      mesh=_MESH,
      compiler_params=pltpu.CompilerParams(use_tc_tiling_on_sc=(D == 128)),
      scratch_types=(
          [pltpu.VMEM((GCB, B), jnp.int32)] * 4
          + [pltpu.VMEM((B, D), jnp.float32)] * (2 * K)
          + [pltpu.VMEM((ZROWS, D), jnp.float32)]
          + [pltpu.SemaphoreType.DMA] * (2 + 4 * K)
          + [pltpu.VMEM_SHARED((NPAD, D), jnp.float32)]
      ),
  )(body)


_scatter128 = _make_scatter_kernel(128, 64, 2, 16)
_scatter64 = _make_scatter_kernel(64, 128, 2, 20)


def _tc1a_body(feat_ref, w1_ref, h_ref):
  h_ref[...] = jnp.dot(feat_ref[...], w1_ref[...],
                       preferred_element_type=jnp.float32)


def _tc1b_body(h_ref, degp_ref, hs_ref, dinv_ref):
  degt = jnp.transpose(degp_ref[...])
  deg = degt[:, 0:1] + degt[:, 1:2] + 1.0
  dinv = lax.rsqrt(deg)
  hs_ref[:N, :] = dinv[:N] * h_ref[...]
  hs_ref[N:, :] = jnp.zeros((NPAD - N, 128), jnp.float32)
  dinv_ref[...] = dinv


def _tc2_body(aggp_ref, hs1_ref, dinv_ref, b1_ref, w2_ref, hs2_ref):
  agg = aggp_ref[0, :, :] + aggp_ref[1, :, :] + hs1_ref[...]
  x1 = jnp.maximum(dinv_ref[...] * agg + b1_ref[...], 0.0)
  h2 = jnp.dot(x1, w2_ref[...], preferred_element_type=jnp.float32)
  hs2_ref[...] = dinv_ref[...] * h2


def _tc3_body(aggp_ref, hs2_ref, dinv_ref, b2_ref, out_ref):
  agg = aggp_ref[0, :, :] + aggp_ref[1, :, :] + hs2_ref[...]
  z = (dinv_ref[...] * agg + b2_ref[...])[:N, :]
  m = jnp.max(z, axis=1, keepdims=True)
  e = jnp.exp(z - m)
  lse = m + jnp.log(jnp.sum(e, axis=1, keepdims=True))
  out_ref[...] = z - lse


def kernel(feature, adj, W1, b1, W2, b2):
  edges = jnp.concatenate(
      [adj.reshape(2, -1, 128), jnp.asarray(_PAD_EDGES)], axis=1)

  h1 = pl.pallas_call(
      _tc1a_body,
      out_shape=jax.ShapeDtypeStruct((N, 128), jnp.float32),
  )(feature, W1)

  deg_parts = _deg_kernel(edges)

  hs1, dinv = pl.pallas_call(
      _tc1b_body,
      out_shape=[
          jax.ShapeDtypeStruct((NPAD, 128), jnp.float32),
          jax.ShapeDtypeStruct((NPAD, 1), jnp.float32),
      ],
  )(h1, deg_parts)

  agg1 = _scatter128(hs1, edges.reshape(2, -1, 64))

  hs2 = pl.pallas_call(
      _tc2_body,
      out_shape=jax.ShapeDtypeStruct((NPAD, 64), jnp.float32),
  )(agg1, hs1, dinv, b1.reshape(1, -1), W2)

  agg2 = _scatter64(hs2, edges)

  out = pl.pallas_call(
      _tc3_body,
      out_shape=jax.ShapeDtypeStruct((N, 64), jnp.float32),
  )(agg2, hs2, dinv, b2.reshape(1, -1))
  return out


# speedup vs baseline: 1.0808x; 1.0114x over previous
import functools

import jax
import jax.numpy as jnp
import numpy as np
from jax import lax
from jax.experimental import pallas as pl
from jax.experimental.pallas import tpu as pltpu
from jax.experimental.pallas import tpu_sc as plsc

N = 10000
NPAD = 10240
NC, NS = 2, 16
EPW = 10240
EPAD = NC * NS * EPW
EROWS = EPAD // 128
RW = EPW // 128
GC = 8
NG = RW // GC
ZW = NPAD // NS
ZROWS = 16

_NPAD_EDGES = EPAD - 320000
_PAD_EDGES = np.stack([
    (np.arange(_NPAD_EDGES, dtype=np.int32) % N).reshape(-1, 128),
    (np.arange(_NPAD_EDGES, dtype=np.int32) % (NPAD - N) + N).reshape(-1, 128),
])

_MESH = plsc.VectorSubcoreMesh(
    core_axis_name="c", subcore_axis_name="s", num_cores=NC, num_subcores=NS)


def _fill_1d(ref, n, value):
  def zf(k, _):
    ref[pl.ds(k * 16, 16)] = jnp.full((16,), value, jnp.float32)
    return 0
  lax.fori_loop(0, n // 16, zf, 0)


def _fill_rows(ref, rows, width, value):
  def zf(k, _):
    r = k // (width // 16)
    col = (k % (width // 16)) * 16
    ref[r, pl.ds(col, 16)] = jnp.full((16,), value, jnp.float32)
    return 0
  lax.fori_loop(0, rows * width // 16, zf, 0)


def _deg_body(edges_hbm, out_hbm, slab_v, ones_v, zbuf_v, dsem, deg_sh):
  c = lax.axis_index("c")
  s = lax.axis_index("s")
  w = c * NS + s
  _fill_1d(ones_v, 128, 1.0)
  _fill_1d(zbuf_v, ZW, 0.0)
  pltpu.sync_copy(edges_hbm.at[1, pl.ds(w * RW, RW)], slab_v)
  pltpu.sync_copy(zbuf_v, deg_sh.at[pl.ds(s * ZW, ZW)])
  plsc.subcore_barrier()
  def step(b, _):
    for k in range(8):
      pltpu.async_copy(ones_v, deg_sh.at[slab_v.at[b * 8 + k]], dsem,
                       add=True)
    for _k in range(8):
      pltpu.make_async_copy(ones_v, deg_sh.at[slab_v.at[0]], dsem).wait()
    return 0
  lax.fori_loop(0, RW // 8, step, 0)
  plsc.subcore_barrier()
  pltpu.sync_copy(deg_sh.at[pl.ds(s * ZW, ZW)],
                  out_hbm.at[c, pl.ds(s * ZW, ZW)])


_deg_kernel = functools.partial(
    pl.kernel,
    out_type=jax.ShapeDtypeStruct((NC, NPAD), jnp.float32),
    mesh=_MESH,
    scratch_types=[
        pltpu.VMEM((RW, 128), jnp.int32),
        pltpu.VMEM((128,), jnp.float32),
        pltpu.VMEM((ZW,), jnp.float32),
        pltpu.SemaphoreType.DMA,
        pltpu.VMEM_SHARED((NPAD,), jnp.float32),
    ],
)(_deg_body)


def _make_scatter_kernel(D, B, K, GCB):
  CH = EPW // B
  NGB = CH // GCB
  assert GCB % (2 * K) == 0

  def body(hs_hbm, idx_hbm, out_hbm, *scratch):
    (si0, di0, si1, di1), scratch = scratch[:4], scratch[4:]
    abufs, scratch = scratch[:K], scratch[K:]
    bbufs, scratch = scratch[:K], scratch[K:]
    (zbuf, isem0, isem1), scratch = scratch[:3], scratch[3:]
    asems, scratch = scratch[:K], scratch[K:]
    bsems, scratch = scratch[:K], scratch[K:]
    sasems, scratch = scratch[:K], scratch[K:]
    sbsems, scratch = scratch[:K], scratch[K:]
    (agg_sh,) = scratch
    c = lax.axis_index("c")
    s = lax.axis_index("s")
    cbase = (c * NS + s) * CH
    zb = s * ZW

    def idx_start(gidx, sbuf, dbuf, sem):
      rows = pl.ds(cbase + gidx * GCB, GCB)
      pltpu.async_copy(idx_hbm.at[0, rows], sbuf, sem)
      pltpu.async_copy(idx_hbm.at[1, rows], dbuf, sem)

    idx_start(0, si0, di0, isem0)
    _fill_rows(zbuf, ZROWS, D, 0.0)
    def zc(t, _):
      pltpu.sync_copy(zbuf, agg_sh.at[pl.ds(zb + t * ZROWS, ZROWS)])
      return 0
    lax.fori_loop(0, ZW // ZROWS, zc, 0)
    plsc.subcore_barrier()

    def idx_wait(sbuf, dbuf, sem):
      pltpu.make_async_copy(idx_hbm.at[0, pl.ds(0, GCB)], sbuf, sem).wait()
      pltpu.make_async_copy(idx_hbm.at[1, pl.ds(0, GCB)], dbuf, sem).wait()

    def gather(row_ref, buf, sem):
      return pltpu.async_copy(hs_hbm.at[row_ref], buf, sem)

    def scat(buf, row_ref, sem):
      return pltpu.async_copy(buf, agg_sh.at[row_ref], sem, add=True)

    def gwait(buf, sem):
      pltpu.make_async_copy(hs_hbm.at[si0.at[0]], buf, sem).wait()

    def prime(sbuf):
      for k in range(K):
        gather(sbuf.at[k], abufs[k], asems[k])

    def block(sbuf, dbuf, j, prefetch):
      for k in range(K):
        gather(sbuf.at[j + K + k], bbufs[k], bsems[k])
      cas = []
      for k in range(K):
        gwait(abufs[k], asems[k])
        cas.append(scat(abufs[k], dbuf.at[j + k], sasems[k]))
      cbs = []
      for k in range(K):
        gwait(bbufs[k], bsems[k])
        cbs.append(scat(bbufs[k], dbuf.at[j + K + k], sbsems[k]))
      for ca in cas:
        ca.wait()
      prefetch(j)
      for cb in cbs:
        cb.wait()

    def group(sbuf, dbuf, nsbuf, ndbuf, nsem):
      def it(t, _):
        block(sbuf, dbuf, 2 * K * t,
              lambda j: prime_rows(sbuf, j + 2 * K))
        return 0
      lax.fori_loop(0, GCB // (2 * K) - 1, it, 0)
      def cross(_j):
        idx_wait(nsbuf, ndbuf, nsem)
        prime(nsbuf)
      block(sbuf, dbuf, GCB - 2 * K, cross)

    def prime_rows(sbuf, j):
      for k in range(K):
        gather(sbuf.at[j + k], abufs[k], asems[k])

    idx_wait(si0, di0, isem0)
    prime(si0)
    def pair(p, _):
      g0 = 2 * p
      idx_start(g0 + 1, si1, di1, isem1)
      group(si0, di0, si1, di1, isem1)
      idx_start(jnp.minimum(g0 + 2, NGB - 1), si0, di0, isem0)
      group(si1, di1, si0, di0, isem0)
      return 0
    lax.fori_loop(0, NGB // 2, pair, 0)
    for k in range(K):
      gwait(abufs[k], asems[k])
    plsc.subcore_barrier()
    pltpu.sync_copy(agg_sh.at[pl.ds(zb, ZW)],
                    out_hbm.at[c, pl.ds(zb, ZW)])

  return functools.partial(
      pl.kernel,
      out_type=jax.ShapeDtypeStruct((NC, NPAD, D), jnp.float32),
      mesh=_MESH,
      compiler_params=pltpu.CompilerParams(use_tc_tiling_on_sc=(D == 128)),
      scratch_types=(
          [pltpu.VMEM((GCB, B), jnp.int32)] * 4
          + [pltpu.VMEM((B, D), jnp.float32)] * (2 * K)
          + [pltpu.VMEM((ZROWS, D), jnp.float32)]
          + [pltpu.SemaphoreType.DMA] * (2 + 4 * K)
          + [pltpu.VMEM_SHARED((NPAD, D), jnp.float32)]
      ),
  )(body)


_scatter128 = _make_scatter_kernel(128, 64, 2, 16)
_scatter64 = _make_scatter_kernel(64, 128, 2, 20)


def _tc1a_body(feat_ref, w1_ref, h_ref):
  h_ref[...] = jnp.dot(feat_ref[...], w1_ref[...],
                       preferred_element_type=jnp.float32)


def _tc1b_body(h_ref, degp_ref, hs_ref, dinv_ref):
  degt = jnp.transpose(degp_ref[...])
  deg = degt[:, 0:1] + degt[:, 1:2] + 1.0
  dinv = lax.rsqrt(deg)
  hs_ref[:N, :] = dinv[:N] * h_ref[...]
  hs_ref[N:, :] = jnp.zeros((NPAD - N, 128), jnp.float32)
  dinv_ref[...] = dinv


def _tc2_body(aggp_ref, hs1_ref, dinv_ref, b1_ref, w2_ref, hs2_ref):
  agg = aggp_ref[0, :, :] + aggp_ref[1, :, :] + hs1_ref[...]
  x1 = jnp.maximum(dinv_ref[...] * agg + b1_ref[...], 0.0)
  h2 = jnp.dot(x1, w2_ref[...], preferred_element_type=jnp.float32)
  hs2_ref[...] = dinv_ref[...] * h2


def _tc3_body(aggp_ref, hs2_ref, dinv_ref, b2_ref, out_ref):
  agg = aggp_ref[0, :, :] + aggp_ref[1, :, :] + hs2_ref[...]
  z = (dinv_ref[...] * agg + b2_ref[...])[:N, :]
  m = jnp.max(z, axis=1, keepdims=True)
  e = jnp.exp(z - m)
  lse = m + jnp.log(jnp.sum(e, axis=1, keepdims=True))
  out_ref[...] = z - lse


def kernel(feature, adj, W1, b1, W2, b2):
  edges = jnp.concatenate(
      [adj.reshape(2, -1, 128), jnp.asarray(_PAD_EDGES)], axis=1)

  h1 = pl.pallas_call(
      _tc1a_body,
      out_shape=jax.ShapeDtypeStruct((N, 128), jnp.float32),
  )(feature, W1)

  deg_parts = _deg_kernel(edges)

  hs1, dinv = pl.pallas_call(
      _tc1b_body,
      out_shape=[
          jax.ShapeDtypeStruct((NPAD, 128), jnp.float32),
          jax.ShapeDtypeStruct((NPAD, 1), jnp.float32),
      ],
  )(h1, deg_parts)

  agg1 = _scatter128(hs1, edges.reshape(2, -1, 64))

  hs2 = pl.pallas_call(
      _tc2_body,
      out_shape=jax.ShapeDtypeStruct((NPAD, 64), jnp.float32),
  )(agg1, hs1, dinv, b1.reshape(1, -1), W2)

  agg2 = _scatter64(hs2, edges)

  out = pl.pallas_call(
      _tc3_body,
      out_shape=jax.ShapeDtypeStruct((N, 64), jnp.float32),
  )(agg2, hs2, dinv, b2.reshape(1, -1))
  return out
